# baseline (speedup 1.0000x reference)
import functools

import jax
import jax.numpy as jnp
from jax import lax
from jax.experimental import pallas as pl
from jax.experimental.pallas import tpu as pltpu
from jax.experimental.pallas import tpu_sc as plsc

N_FACTORS = 32
BATCH = 16384
N_ROWS = 1000000
PACK = 4
PACKED_W = N_FACTORS * PACK
NC = 2
NS = 16
L = 16
NW = NC * NS
B_PER_W = BATCH // NW
IDX_CHUNK = 128
N_CHUNKS = B_PER_W // IDX_CHUNK


def _body(user_hbm, item_hbm, uf_hbm, if_hbm, out_hbm,
          uidx_v, iidx_v, ubase_v, ibase_v, ubuf, ibuf, out_v, sem):
    wid = lax.axis_index("s") * NC + lax.axis_index("c")
    base = wid * B_PER_W

    for j in range(N_CHUNKS):
        hsl = pl.ds(base + j * IDX_CHUNK, IDX_CHUNK)
        pltpu.sync_copy(user_hbm.at[hsl], uidx_v.at[j])
        pltpu.sync_copy(item_hbm.at[hsl], iidx_v.at[j])

    for j in range(N_CHUNKS):
        for i in range(IDX_CHUNK // L):
            sl = (j, pl.ds(i * L, L))
            u = uidx_v[sl] - 1
            ubase_v[sl] = (u & 3) * N_FACTORS
            uidx_v[sl] = lax.shift_right_logical(u, 2)
            it = iidx_v[sl] - 1
            ibase_v[sl] = (it & 3) * N_FACTORS
            iidx_v[sl] = lax.shift_right_logical(it, 2)

    lanes = lax.iota(jnp.int32, L)

    for j in range(N_CHUNKS):
        cu = pltpu.async_copy(uf_hbm.at[uidx_v.at[j]], ubuf.at[...], sem)
        ci = pltpu.async_copy(if_hbm.at[iidx_v.at[j]], ibuf.at[...], sem)
        cu.wait()
        ci.wait()

        def group(g, carry, j=j):
            rsl = pl.ds(g * L, L)
            rows = g * L + lanes
            ub = ubase_v[(j, rsl)]
            ib = ibase_v[(j, rsl)]
            acc = jnp.zeros((L,), jnp.float32)
            for d in range(N_FACTORS):
                uv = plsc.load_gather(ubuf, [rows, ub + d])
                iv = plsc.load_gather(ibuf, [rows, ib + d])
                acc = acc + uv * iv
            out_v[pl.ds(j * IDX_CHUNK + g * L, L)] = acc * 5.0
            return carry

        lax.fori_loop(0, IDX_CHUNK // L, group, 0)

    pltpu.sync_copy(out_v.at[...], out_hbm.at[pl.ds(base, B_PER_W)])


@jax.jit
def _mf_forward(user, item, uf_packed, if_packed):
    mesh = plsc.VectorSubcoreMesh(core_axis_name="c", subcore_axis_name="s")
    f = pl.kernel(
        _body,
        mesh=mesh,
        out_type=jax.ShapeDtypeStruct((BATCH,), jnp.float32),
        scratch_types=[
            pltpu.VMEM((N_CHUNKS, IDX_CHUNK), jnp.int32),
            pltpu.VMEM((N_CHUNKS, IDX_CHUNK), jnp.int32),
            pltpu.VMEM((N_CHUNKS, IDX_CHUNK), jnp.int32),
            pltpu.VMEM((N_CHUNKS, IDX_CHUNK), jnp.int32),
            pltpu.VMEM((IDX_CHUNK, PACKED_W), jnp.float32),
            pltpu.VMEM((IDX_CHUNK, PACKED_W), jnp.float32),
            pltpu.VMEM((B_PER_W,), jnp.float32),
            pltpu.SemaphoreType.DMA,
        ],
        compiler_params=pltpu.CompilerParams(needs_layout_passes=False),
    )
    return f(user, item, uf_packed, if_packed)


def kernel(user, item, user_factors, item_factors):
    uf_packed = user_factors.reshape(N_ROWS // PACK, PACKED_W)
    if_packed = item_factors.reshape(N_ROWS // PACK, PACKED_W)
    return _mf_forward(user, item, uf_packed, if_packed)
